# baseline (speedup 1.0000x reference)
import functools

import jax
import jax.numpy as jnp
import numpy as np
from jax.experimental import pallas as pl
from jax.experimental.pallas import tpu as pltpu

_K = 16
_RATIO = 0.25


def _rup(x, m):
    return (x + m - 1) // m * m


def _knn_body(qT_ref, dT_ref, dsq_ref, out_ref, *, k, exclude_self, blk_q):
    qT = qT_ref[...]
    dT = dT_ref[...]
    cross = jax.lax.dot_general(qT, dT, (((0,), (0,)), ((), ())),
                                preferred_element_type=jnp.float32)
    dist = dsq_ref[...] - 2.0 * cross
    col = jax.lax.broadcasted_iota(jnp.int32, dist.shape, 1)
    if exclude_self:
        row0 = pl.program_id(0) * blk_q
        rows = row0 + jax.lax.broadcasted_iota(jnp.int32, dist.shape, 0)
        dist = jnp.where(col == rows, jnp.float32(np.inf), dist)
    big_i = jnp.int32(2**30)
    for j in range(k):
        m = jnp.min(dist, axis=1, keepdims=True)
        idx = jnp.min(jnp.where(dist == m, col, big_i), axis=1)
        out_ref[:, j] = idx.astype(jnp.int32)
        dist = jnp.where(col == idx[:, None], jnp.float32(np.inf), dist)


def _knn(qpos, dpos, k, exclude_self):
    nq, nd = qpos.shape[0], dpos.shape[0]
    blk = min(256, _rup(nq, 8))
    nq_pad = _rup(nq, blk)
    nd_pad = _rup(nd, 128)
    qT = jnp.zeros((3, nq_pad), jnp.float32).at[:, :nq].set(qpos.T)
    dT = jnp.zeros((3, nd_pad), jnp.float32).at[:, :nd].set(dpos.T)
    dsq = jnp.full((1, nd_pad), 1e30, jnp.float32)
    dsq = dsq.at[0, :nd].set(jnp.sum(dpos * dpos, -1))
    out = pl.pallas_call(
        functools.partial(_knn_body, k=k, exclude_self=exclude_self, blk_q=blk),
        grid=(nq_pad // blk,),
        in_specs=[
            pl.BlockSpec((3, blk), lambda i: (0, i)),
            pl.BlockSpec((3, nd_pad), lambda i: (0, 0)),
            pl.BlockSpec((1, nd_pad), lambda i: (0, 0)),
        ],
        out_specs=pl.BlockSpec((blk, k), lambda i: (i, 0)),
        out_shape=jax.ShapeDtypeStruct((nq_pad, k), jnp.int32),
    )(qT, dT, dsq)
    return out[:nq]


def _fps_body(pos_ref, out_ref, *, m, n, rows, orows):
    pall = pos_ref[...]
    px, py, pz = pall[0], pall[1], pall[2]
    flat = (jax.lax.broadcasted_iota(jnp.int32, (rows, 128), 0) * 128
            + jax.lax.broadcasted_iota(jnp.int32, (rows, 128), 1))
    oflat = (jax.lax.broadcasted_iota(jnp.int32, (orows, 128), 0) * 128
             + jax.lax.broadcasted_iota(jnp.int32, (orows, 128), 1))
    valid = flat < n
    big_i = jnp.int32(2**30)

    def dist_to(ix):
        sel = flat == ix
        sx = jnp.sum(jnp.where(sel, px, 0.0))
        sy = jnp.sum(jnp.where(sel, py, 0.0))
        sz = jnp.sum(jnp.where(sel, pz, 0.0))
        dx = px - sx
        dy = py - sy
        dz = pz - sz
        return dx * dx + dy * dy + dz * dz

    mind = jnp.where(valid, dist_to(jnp.int32(0)), jnp.float32(-1.0))
    outarr = jnp.zeros((orows, 128), jnp.int32)

    def body(i, st):
        mind, outarr = st
        mx = jnp.max(mind)
        nxt = jnp.min(jnp.where(mind == mx, flat, big_i)).astype(jnp.int32)
        outarr = jnp.where(oflat == i, nxt, outarr)
        return jnp.minimum(mind, dist_to(nxt)), outarr

    _, outarr = jax.lax.fori_loop(1, m, body, (mind, outarr))
    out_ref[...] = outarr


def _fps(pos, m):
    n = pos.shape[0]
    rows = _rup((n + 127) // 128, 8)
    pad = jnp.zeros((3, rows * 128), jnp.float32).at[:, :n].set(pos.T)
    pad = pad.reshape(3, rows, 128)
    orows = _rup((m + 127) // 128, 8)
    out = pl.pallas_call(
        functools.partial(_fps_body, m=m, n=n, rows=rows, orows=orows),
        out_shape=jax.ShapeDtypeStruct((orows, 128), jnp.int32),
    )(pad)
    return out.reshape(-1)[:m]


def _proj_body(x_ref, win_ref, bin_ref, wcat_ref, out_ref):
    h = jnp.maximum(
        jax.lax.dot_general(x_ref[...], win_ref[...], (((1,), (1,)), ((), ())),
                            preferred_element_type=jnp.float32) + bin_ref[...], 0.0)
    out_ref[...] = jax.lax.dot_general(h, wcat_ref[...], (((1,), (1,)), ((), ())),
                                       preferred_element_type=jnp.float32)


def _conv_body(adst_ref, aself_ref, xlself_ref, gsrc_ref, gxl_ref, gpos_ref,
               posq_ref, pw1_ref, pw2_ref, aw1_ref, aw2_ref, wout_ref, bout_ref,
               out_ref, *, blk, dout, kk):
    kp1 = kk + 1
    posq = posq_ref[...]
    rel = posq[:, None, :] - jnp.concatenate(
        [gpos_ref[...], posq[:, None, :]], 1)
    pw1t = pw1_ref[...]
    h = (rel[:, :, 0:1] * pw1t[0].reshape(1, 1, 64)
         + rel[:, :, 1:2] * pw1t[1].reshape(1, 1, 64)
         + rel[:, :, 2:3] * pw1t[2].reshape(1, 1, 64))
    h = jnp.maximum(h, 0.0).reshape(blk * kp1, 64)
    delta = jnp.maximum(jax.lax.dot_general(h, pw2_ref[...], (((1,), (1,)), ((), ())),
                                            preferred_element_type=jnp.float32), 0.0)
    delta3 = delta.reshape(blk, kp1, dout)
    asrc_full = jnp.concatenate([gsrc_ref[...], aself_ref[...][:, None, :]], 1)
    ain = adst_ref[...][:, None, :] - asrc_full + delta3
    h2 = jnp.maximum(jax.lax.dot_general(ain.reshape(blk * kp1, dout), aw1_ref[...],
                                         (((1,), (1,)), ((), ())),
                                         preferred_element_type=jnp.float32), 0.0)
    alpha = jnp.maximum(jax.lax.dot_general(h2, aw2_ref[...], (((1,), (1,)), ((), ())),
                                            preferred_element_type=jnp.float32),
                        0.0).reshape(blk, kp1, dout)
    amax = jnp.max(alpha, axis=1, keepdims=True)
    ex = jnp.exp(alpha - amax)
    den = jnp.sum(ex, axis=1, keepdims=True)
    attn = ex / (den + 1e-16)
    xl_full = jnp.concatenate([gxl_ref[...], xlself_ref[...][:, None, :]], 1) + delta3
    o = jnp.sum(attn * xl_full, axis=1)
    o = jnp.maximum(jax.lax.dot_general(o, wout_ref[...], (((1,), (1,)), ((), ())),
                                        preferred_element_type=jnp.float32)
                    + bout_ref[...], 0.0)
    out_ref[...] = o


def _block_fused(p, x, pos, nbr):
    n, din = x.shape
    c = p["conv"]
    dout = c["lin"].shape[0]
    kk = nbr.shape[1]
    blk = max(64, 32768 // dout)
    n_pad = _rup(n, blk)
    xp = jnp.zeros((n_pad, din), jnp.float32).at[:n].set(x)
    wcat = jnp.concatenate([c["lin"], c["lin_src"], c["lin_dst"]], 0)
    proj = pl.pallas_call(
        _proj_body,
        grid=(n_pad // blk,),
        in_specs=[
            pl.BlockSpec((blk, din), lambda i: (i, 0)),
            pl.BlockSpec((din, din), lambda i: (0, 0)),
            pl.BlockSpec((1, din), lambda i: (0, 0)),
            pl.BlockSpec((3 * dout, din), lambda i: (0, 0)),
        ],
        out_specs=pl.BlockSpec((blk, 3 * dout), lambda i: (i, 0)),
        out_shape=jax.ShapeDtypeStruct((n_pad, 3 * dout), jnp.float32),
    )(xp, p["lin_in"]["w"], p["lin_in"]["b"][None, :], wcat)
    xl = proj[:, :dout]
    a_src = proj[:, dout:2 * dout]
    a_dst = proj[:, 2 * dout:]
    nbr_p = jnp.zeros((n_pad, kk), nbr.dtype).at[:n].set(nbr)
    gsrc = a_src[nbr_p]
    gxl = xl[nbr_p]
    posp = jnp.zeros((n_pad, 3), jnp.float32).at[:n].set(pos)
    gpos = posp[nbr_p]
    pn0, pn1 = c["pos_nn"][0]["lin"], c["pos_nn"][1]["lin"]
    an0, an1 = c["attn_nn"][0]["lin"], c["attn_nn"][1]["lin"]
    out = pl.pallas_call(
        functools.partial(_conv_body, blk=blk, dout=dout, kk=kk),
        grid=(n_pad // blk,),
        in_specs=[
            pl.BlockSpec((blk, dout), lambda i: (i, 0)),
            pl.BlockSpec((blk, dout), lambda i: (i, 0)),
            pl.BlockSpec((blk, dout), lambda i: (i, 0)),
            pl.BlockSpec((blk, kk, dout), lambda i: (i, 0, 0)),
            pl.BlockSpec((blk, kk, dout), lambda i: (i, 0, 0)),
            pl.BlockSpec((blk, kk, 3), lambda i: (i, 0, 0)),
            pl.BlockSpec((blk, 3), lambda i: (i, 0)),
            pl.BlockSpec((3, 64), lambda i: (0, 0)),
            pl.BlockSpec((dout, 64), lambda i: (0, 0)),
            pl.BlockSpec((64, dout), lambda i: (0, 0)),
            pl.BlockSpec((dout, 64), lambda i: (0, 0)),
            pl.BlockSpec((dout, dout), lambda i: (0, 0)),
            pl.BlockSpec((1, dout), lambda i: (0, 0)),
        ],
        out_specs=pl.BlockSpec((blk, dout), lambda i: (i, 0)),
        out_shape=jax.ShapeDtypeStruct((n_pad, dout), jnp.float32),
    )(a_dst, a_src, xl, gsrc, gxl, gpos, posp,
      pn0["w"].T, pn1["w"], an0["w"], an1["w"],
      p["lin_out"]["w"], p["lin_out"]["b"][None, :])
    return out[:n]


def _lin(p, x):
    return x @ p["w"].T + p["b"]


def _bn(p, x):
    mu = jnp.mean(x, 0)
    var = jnp.var(x, 0)
    return p["gamma"] * (x - mu) / jnp.sqrt(var + 1e-5) + p["beta"]


def _mlp_bn(ps, x):
    for p in ps:
        x = jax.nn.relu(_bn(p["bn"], _lin(p["lin"], x)))
    return x


def _mlp_nobn(ps, x):
    for p in ps:
        x = jax.nn.relu(_lin(p["lin"], x))
    return x


def _conv_dense(p, x, pos, nbr):
    n = x.shape[0]
    nbr_full = jnp.concatenate([nbr, jnp.arange(n, dtype=nbr.dtype)[:, None]], 1)
    xl = x @ p["lin"].T
    a_src = x @ p["lin_src"].T
    a_dst = x @ p["lin_dst"].T
    rel = pos[:, None, :] - pos[nbr_full]
    delta = _mlp_nobn(p["pos_nn"], rel)
    alpha = _mlp_nobn(p["attn_nn"], a_dst[:, None, :] - a_src[nbr_full] + delta)
    amax = jnp.max(alpha, axis=1, keepdims=True)
    ex = jnp.exp(alpha - amax)
    den = jnp.sum(ex, axis=1, keepdims=True)
    attn = ex / (den + 1e-16)
    return jnp.sum(attn * (xl[nbr_full] + delta), axis=1)


def _block(p, x, pos, nbr):
    return _block_fused(p, x, pos, nbr)


def _interp(x_sub, pos_sub, pos, k=3):
    nbr = _knn(pos, pos_sub, k, exclude_self=False)
    diff = pos_sub[nbr] - pos[:, None, :]
    sq = jnp.sum(diff * diff, -1, keepdims=True)
    w = 1.0 / jnp.maximum(sq, 1e-16)
    return jnp.sum(x_sub[nbr] * w, axis=1) / jnp.sum(w, axis=1)


def kernel(x, pos, params):
    n0 = pos.shape[0]
    x = _mlp_bn(params["mlp_input"], x)
    nbr0 = _knn(pos, pos, _K, exclude_self=True)
    x = _block(params["transformer_input"], x, pos, nbr0)

    xs, poss, nbrs = [x], [pos], [nbr0]
    for enc in params["encoders"]:
        cur_pos = poss[-1]
        m = int(np.ceil(cur_pos.shape[0] * _RATIO))
        idc = _fps(cur_pos, m)
        nbr_dn = _knn(cur_pos[idc], cur_pos, _K, exclude_self=False)
        xh = _mlp_bn(enc["down"]["mlp"], xs[-1])
        x = jnp.max(xh[nbr_dn], axis=1)
        pos_new = cur_pos[idc]
        nbr = _knn(pos_new, pos_new, _K, exclude_self=True)
        x = _block(enc["block"], x, pos_new, nbr)
        xs.append(x)
        poss.append(pos_new)
        nbrs.append(nbr)

    x = _mlp_nobn(params["mlp_summit"], xs[-1])
    x = _block(params["transformer_summit"], x, poss[-1], nbrs[-1])

    for i, dec in enumerate(params["decoders"]):
        x_skip = xs[-i - 2]
        pos_f, pos_c = poss[-i - 2], poss[-i - 1]
        x_sub = _mlp_bn(dec["up"]["mlp_sub"], x)
        xi = _interp(x_sub, pos_c, pos_f, k=3)
        x = _mlp_bn(dec["up"]["mlp"], x_skip) + xi
        x = _block(dec["block"], x, pos_f, nbrs[-i - 2])

    p0, p1, p2 = params["mlp_output"]
    x = jax.nn.relu(_lin(p0, x))
    x = jax.nn.relu(_lin(p1, x))
    x = _lin(p2, x)
    return jax.nn.log_softmax(x, axis=-1)


# speedup vs baseline: 1.1615x; 1.1615x over previous
import functools

import jax
import jax.numpy as jnp
import numpy as np
from jax.experimental import pallas as pl
from jax.experimental.pallas import tpu as pltpu

_K = 16
_RATIO = 0.25


def _rup(x, m):
    return (x + m - 1) // m * m


def _knn_body(qT_ref, dT_ref, dsq_ref, out_ref, *, k, exclude_self, blk_q):
    qT = qT_ref[...]
    dT = dT_ref[...]
    cross = jax.lax.dot_general(qT, dT, (((0,), (0,)), ((), ())),
                                preferred_element_type=jnp.float32)
    dist = dsq_ref[...] - 2.0 * cross
    col = jax.lax.broadcasted_iota(jnp.int32, dist.shape, 1)
    if exclude_self:
        row0 = pl.program_id(0) * blk_q
        rows = row0 + jax.lax.broadcasted_iota(jnp.int32, dist.shape, 0)
        dist = jnp.where(col == rows, jnp.float32(np.inf), dist)
    big_i = jnp.int32(2**30)
    for j in range(k):
        m = jnp.min(dist, axis=1, keepdims=True)
        idx = jnp.min(jnp.where(dist == m, col, big_i), axis=1)
        out_ref[:, j] = idx.astype(jnp.int32)
        dist = jnp.where(col == idx[:, None], jnp.float32(np.inf), dist)


def _knn(qpos, dpos, k, exclude_self):
    nq, nd = qpos.shape[0], dpos.shape[0]
    blk = min(256, _rup(nq, 8))
    nq_pad = _rup(nq, blk)
    nd_pad = _rup(nd, 128)
    qT = jnp.zeros((3, nq_pad), jnp.float32).at[:, :nq].set(qpos.T)
    dT = jnp.zeros((3, nd_pad), jnp.float32).at[:, :nd].set(dpos.T)
    dsq = jnp.full((1, nd_pad), 1e30, jnp.float32)
    dsq = dsq.at[0, :nd].set(jnp.sum(dpos * dpos, -1))
    out = pl.pallas_call(
        functools.partial(_knn_body, k=k, exclude_self=exclude_self, blk_q=blk),
        grid=(nq_pad // blk,),
        in_specs=[
            pl.BlockSpec((3, blk), lambda i: (0, i)),
            pl.BlockSpec((3, nd_pad), lambda i: (0, 0)),
            pl.BlockSpec((1, nd_pad), lambda i: (0, 0)),
        ],
        out_specs=pl.BlockSpec((blk, k), lambda i: (i, 0)),
        out_shape=jax.ShapeDtypeStruct((nq_pad, k), jnp.int32),
    )(qT, dT, dsq)
    return out[:nq]


def _fps_body(pos_ref, out_ref, *, m, n, rows, orows):
    pall = pos_ref[...]
    px, py, pz = pall[0], pall[1], pall[2]
    flat = (jax.lax.broadcasted_iota(jnp.int32, (rows, 128), 0) * 128
            + jax.lax.broadcasted_iota(jnp.int32, (rows, 128), 1))
    oflat = (jax.lax.broadcasted_iota(jnp.int32, (orows, 128), 0) * 128
             + jax.lax.broadcasted_iota(jnp.int32, (orows, 128), 1))
    valid = flat < n
    big_i = jnp.int32(2**30)

    def dist_to(ix):
        sel = flat == ix
        sx = jnp.sum(jnp.where(sel, px, 0.0))
        sy = jnp.sum(jnp.where(sel, py, 0.0))
        sz = jnp.sum(jnp.where(sel, pz, 0.0))
        dx = px - sx
        dy = py - sy
        dz = pz - sz
        return dx * dx + dy * dy + dz * dz

    mind = jnp.where(valid, dist_to(jnp.int32(0)), jnp.float32(-1.0))
    outarr = jnp.zeros((orows, 128), jnp.int32)

    def body(i, st):
        mind, outarr = st
        mx = jnp.max(mind)
        nxt = jnp.min(jnp.where(mind == mx, flat, big_i)).astype(jnp.int32)
        outarr = jnp.where(oflat == i, nxt, outarr)
        return jnp.minimum(mind, dist_to(nxt)), outarr

    _, outarr = jax.lax.fori_loop(1, m, body, (mind, outarr))
    out_ref[...] = outarr


def _fps(pos, m):
    n = pos.shape[0]
    rows = _rup((n + 127) // 128, 8)
    pad = jnp.zeros((3, rows * 128), jnp.float32).at[:, :n].set(pos.T)
    pad = pad.reshape(3, rows, 128)
    orows = _rup((m + 127) // 128, 8)
    out = pl.pallas_call(
        functools.partial(_fps_body, m=m, n=n, rows=rows, orows=orows),
        out_shape=jax.ShapeDtypeStruct((orows, 128), jnp.int32),
    )(pad)
    return out.reshape(-1)[:m]


def _proj_body(x_ref, win_ref, bin_ref, wcat_ref, out_ref):
    h = jnp.maximum(
        jax.lax.dot_general(x_ref[...], win_ref[...], (((1,), (1,)), ((), ())),
                            preferred_element_type=jnp.float32) + bin_ref[...], 0.0)
    out_ref[...] = jax.lax.dot_general(h, wcat_ref[...], (((1,), (1,)), ((), ())),
                                       preferred_element_type=jnp.float32)


def _conv_body(adst_ref, aself_ref, xlself_ref, gsrc_ref, gxl_ref, gpos_ref,
               posq_ref, pw1_ref, pw2_ref, aw1_ref, aw2_ref, wout_ref, bout_ref,
               out_ref, *, blk, dout, kk):
    posq = posq_ref[...]
    rel = posq[:, None, :] - gpos_ref[...]
    pw1t = pw1_ref[...]
    h = (rel[:, :, 0:1] * pw1t[0].reshape(1, 1, 64)
         + rel[:, :, 1:2] * pw1t[1].reshape(1, 1, 64)
         + rel[:, :, 2:3] * pw1t[2].reshape(1, 1, 64))
    h = jnp.maximum(h, 0.0).reshape(blk * kk, 64)
    delta = jnp.maximum(jax.lax.dot_general(h, pw2_ref[...], (((1,), (1,)), ((), ())),
                                            preferred_element_type=jnp.float32), 0.0)
    delta3 = delta.reshape(blk, kk, dout)
    adst = adst_ref[...]
    ain = adst[:, None, :] - gsrc_ref[...] + delta3
    h2 = jnp.maximum(jax.lax.dot_general(ain.reshape(blk * kk, dout), aw1_ref[...],
                                         (((1,), (1,)), ((), ())),
                                         preferred_element_type=jnp.float32), 0.0)
    alpha = jnp.maximum(jax.lax.dot_general(h2, aw2_ref[...], (((1,), (1,)), ((), ())),
                                            preferred_element_type=jnp.float32),
                        0.0).reshape(blk, kk, dout)
    sin = adst - aself_ref[...]
    hs = jnp.maximum(jax.lax.dot_general(sin, aw1_ref[...], (((1,), (1,)), ((), ())),
                                         preferred_element_type=jnp.float32), 0.0)
    alpha_s = jnp.maximum(jax.lax.dot_general(hs, aw2_ref[...], (((1,), (1,)), ((), ())),
                                              preferred_element_type=jnp.float32), 0.0)
    amax = jnp.maximum(jnp.max(alpha, axis=1), alpha_s)
    ex = jnp.exp(alpha - amax[:, None, :])
    ex_s = jnp.exp(alpha_s - amax)
    inv = 1.0 / (jnp.sum(ex, axis=1) + ex_s + 1e-16)
    attn = ex * inv[:, None, :]
    o = (jnp.sum(attn * (gxl_ref[...] + delta3), axis=1)
         + (ex_s * inv) * xlself_ref[...])
    o = jnp.maximum(jax.lax.dot_general(o, wout_ref[...], (((1,), (1,)), ((), ())),
                                        preferred_element_type=jnp.float32)
                    + bout_ref[...], 0.0)
    out_ref[...] = o


def _block_fused(p, x, pos, nbr):
    n, din = x.shape
    c = p["conv"]
    dout = c["lin"].shape[0]
    kk = nbr.shape[1]
    blk = max(64, 32768 // dout)
    n_pad = _rup(n, blk)
    xp = jnp.zeros((n_pad, din), jnp.float32).at[:n].set(x)
    wcat = jnp.concatenate([c["lin"], c["lin_src"], c["lin_dst"]], 0)
    proj = pl.pallas_call(
        _proj_body,
        grid=(n_pad // blk,),
        in_specs=[
            pl.BlockSpec((blk, din), lambda i: (i, 0)),
            pl.BlockSpec((din, din), lambda i: (0, 0)),
            pl.BlockSpec((1, din), lambda i: (0, 0)),
            pl.BlockSpec((3 * dout, din), lambda i: (0, 0)),
        ],
        out_specs=pl.BlockSpec((blk, 3 * dout), lambda i: (i, 0)),
        out_shape=jax.ShapeDtypeStruct((n_pad, 3 * dout), jnp.float32),
    )(xp, p["lin_in"]["w"], p["lin_in"]["b"][None, :], wcat)
    xl = proj[:, :dout]
    a_src = proj[:, dout:2 * dout]
    a_dst = proj[:, 2 * dout:]
    nbr_p = jnp.zeros((n_pad, kk), nbr.dtype).at[:n].set(nbr)
    gsrc = a_src[nbr_p]
    gxl = xl[nbr_p]
    posp = jnp.zeros((n_pad, 3), jnp.float32).at[:n].set(pos)
    gpos = posp[nbr_p]
    pn0, pn1 = c["pos_nn"][0]["lin"], c["pos_nn"][1]["lin"]
    an0, an1 = c["attn_nn"][0]["lin"], c["attn_nn"][1]["lin"]
    out = pl.pallas_call(
        functools.partial(_conv_body, blk=blk, dout=dout, kk=kk),
        grid=(n_pad // blk,),
        in_specs=[
            pl.BlockSpec((blk, dout), lambda i: (i, 0)),
            pl.BlockSpec((blk, dout), lambda i: (i, 0)),
            pl.BlockSpec((blk, dout), lambda i: (i, 0)),
            pl.BlockSpec((blk, kk, dout), lambda i: (i, 0, 0)),
            pl.BlockSpec((blk, kk, dout), lambda i: (i, 0, 0)),
            pl.BlockSpec((blk, kk, 3), lambda i: (i, 0, 0)),
            pl.BlockSpec((blk, 3), lambda i: (i, 0)),
            pl.BlockSpec((3, 64), lambda i: (0, 0)),
            pl.BlockSpec((dout, 64), lambda i: (0, 0)),
            pl.BlockSpec((64, dout), lambda i: (0, 0)),
            pl.BlockSpec((dout, 64), lambda i: (0, 0)),
            pl.BlockSpec((dout, dout), lambda i: (0, 0)),
            pl.BlockSpec((1, dout), lambda i: (0, 0)),
        ],
        out_specs=pl.BlockSpec((blk, dout), lambda i: (i, 0)),
        out_shape=jax.ShapeDtypeStruct((n_pad, dout), jnp.float32),
    )(a_dst, a_src, xl, gsrc, gxl, gpos, posp,
      pn0["w"].T, pn1["w"], an0["w"], an1["w"],
      p["lin_out"]["w"], p["lin_out"]["b"][None, :])
    return out[:n]


def _lin(p, x):
    return x @ p["w"].T + p["b"]


def _bn(p, x):
    mu = jnp.mean(x, 0)
    var = jnp.var(x, 0)
    return p["gamma"] * (x - mu) / jnp.sqrt(var + 1e-5) + p["beta"]


def _mlp_bn(ps, x):
    for p in ps:
        x = jax.nn.relu(_bn(p["bn"], _lin(p["lin"], x)))
    return x


def _mlp_nobn(ps, x):
    for p in ps:
        x = jax.nn.relu(_lin(p["lin"], x))
    return x


def _conv_dense(p, x, pos, nbr):
    n = x.shape[0]
    nbr_full = jnp.concatenate([nbr, jnp.arange(n, dtype=nbr.dtype)[:, None]], 1)
    xl = x @ p["lin"].T
    a_src = x @ p["lin_src"].T
    a_dst = x @ p["lin_dst"].T
    rel = pos[:, None, :] - pos[nbr_full]
    delta = _mlp_nobn(p["pos_nn"], rel)
    alpha = _mlp_nobn(p["attn_nn"], a_dst[:, None, :] - a_src[nbr_full] + delta)
    amax = jnp.max(alpha, axis=1, keepdims=True)
    ex = jnp.exp(alpha - amax)
    den = jnp.sum(ex, axis=1, keepdims=True)
    attn = ex / (den + 1e-16)
    return jnp.sum(attn * (xl[nbr_full] + delta), axis=1)


def _block(p, x, pos, nbr):
    return _block_fused(p, x, pos, nbr)


def _interp(x_sub, pos_sub, pos, k=3):
    nbr = _knn(pos, pos_sub, k, exclude_self=False)
    diff = pos_sub[nbr] - pos[:, None, :]
    sq = jnp.sum(diff * diff, -1, keepdims=True)
    w = 1.0 / jnp.maximum(sq, 1e-16)
    return jnp.sum(x_sub[nbr] * w, axis=1) / jnp.sum(w, axis=1)


def kernel(x, pos, params):
    n0 = pos.shape[0]
    x = _mlp_bn(params["mlp_input"], x)
    nbr0 = _knn(pos, pos, _K, exclude_self=True)
    x = _block(params["transformer_input"], x, pos, nbr0)

    xs, poss, nbrs = [x], [pos], [nbr0]
    for enc in params["encoders"]:
        cur_pos = poss[-1]
        m = int(np.ceil(cur_pos.shape[0] * _RATIO))
        idc = _fps(cur_pos, m)
        nbr_dn = _knn(cur_pos[idc], cur_pos, _K, exclude_self=False)
        xh = _mlp_bn(enc["down"]["mlp"], xs[-1])
        x = jnp.max(xh[nbr_dn], axis=1)
        pos_new = cur_pos[idc]
        nbr = _knn(pos_new, pos_new, _K, exclude_self=True)
        x = _block(enc["block"], x, pos_new, nbr)
        xs.append(x)
        poss.append(pos_new)
        nbrs.append(nbr)

    x = _mlp_nobn(params["mlp_summit"], xs[-1])
    x = _block(params["transformer_summit"], x, poss[-1], nbrs[-1])

    for i, dec in enumerate(params["decoders"]):
        x_skip = xs[-i - 2]
        pos_f, pos_c = poss[-i - 2], poss[-i - 1]
        x_sub = _mlp_bn(dec["up"]["mlp_sub"], x)
        xi = _interp(x_sub, pos_c, pos_f, k=3)
        x = _mlp_bn(dec["up"]["mlp"], x_skip) + xi
        x = _block(dec["block"], x, pos_f, nbrs[-i - 2])

    p0, p1, p2 = params["mlp_output"]
    x = jax.nn.relu(_lin(p0, x))
    x = jax.nn.relu(_lin(p1, x))
    x = _lin(p2, x)
    return jax.nn.log_softmax(x, axis=-1)


# speedup vs baseline: 1.1687x; 1.0063x over previous
import functools

import jax
import jax.numpy as jnp
import numpy as np
from jax.experimental import pallas as pl
from jax.experimental.pallas import tpu as pltpu

_K = 16
_RATIO = 0.25


def _rup(x, m):
    return (x + m - 1) // m * m


def _knn_body(qT_ref, dT_ref, dsq_ref, out_ref, *, k, exclude_self, blk_q):
    qT = qT_ref[...]
    dT = dT_ref[...]
    cross = jax.lax.dot_general(qT, dT, (((0,), (0,)), ((), ())),
                                preferred_element_type=jnp.float32)
    dist = dsq_ref[...] - 2.0 * cross
    col = jax.lax.broadcasted_iota(jnp.int32, dist.shape, 1)
    if exclude_self:
        row0 = pl.program_id(0) * blk_q
        rows = row0 + jax.lax.broadcasted_iota(jnp.int32, dist.shape, 0)
        dist = jnp.where(col == rows, jnp.float32(np.inf), dist)
    big_i = jnp.int32(2**30)
    for j in range(k):
        m = jnp.min(dist, axis=1, keepdims=True)
        idx = jnp.min(jnp.where(dist == m, col, big_i), axis=1)
        out_ref[:, j] = idx.astype(jnp.int32)
        dist = jnp.where(col == idx[:, None], jnp.float32(np.inf), dist)


def _knn(qpos, dpos, k, exclude_self):
    nq, nd = qpos.shape[0], dpos.shape[0]
    blk = min(256, _rup(nq, 8))
    nq_pad = _rup(nq, blk)
    nd_pad = _rup(nd, 128)
    qT = jnp.zeros((3, nq_pad), jnp.float32).at[:, :nq].set(qpos.T)
    dT = jnp.zeros((3, nd_pad), jnp.float32).at[:, :nd].set(dpos.T)
    dsq = jnp.full((1, nd_pad), 1e30, jnp.float32)
    dsq = dsq.at[0, :nd].set(jnp.sum(dpos * dpos, -1))
    out = pl.pallas_call(
        functools.partial(_knn_body, k=k, exclude_self=exclude_self, blk_q=blk),
        grid=(nq_pad // blk,),
        in_specs=[
            pl.BlockSpec((3, blk), lambda i: (0, i)),
            pl.BlockSpec((3, nd_pad), lambda i: (0, 0)),
            pl.BlockSpec((1, nd_pad), lambda i: (0, 0)),
        ],
        out_specs=pl.BlockSpec((blk, k), lambda i: (i, 0)),
        out_shape=jax.ShapeDtypeStruct((nq_pad, k), jnp.int32),
    )(qT, dT, dsq)
    return out[:nq]


def _fps_body(pos_ref, out_ref, *, m, n, rows, orows):
    pall = pos_ref[...]
    px, py, pz = pall[0], pall[1], pall[2]
    flat = (jax.lax.broadcasted_iota(jnp.int32, (rows, 128), 0) * 128
            + jax.lax.broadcasted_iota(jnp.int32, (rows, 128), 1))
    oflat = (jax.lax.broadcasted_iota(jnp.int32, (orows, 128), 0) * 128
             + jax.lax.broadcasted_iota(jnp.int32, (orows, 128), 1))
    valid = flat < n
    big_i = jnp.int32(2**30)

    def dist_to(ix):
        sel = flat == ix
        sx = jnp.sum(jnp.where(sel, px, 0.0))
        sy = jnp.sum(jnp.where(sel, py, 0.0))
        sz = jnp.sum(jnp.where(sel, pz, 0.0))
        dx = px - sx
        dy = py - sy
        dz = pz - sz
        return dx * dx + dy * dy + dz * dz

    mind = jnp.where(valid, dist_to(jnp.int32(0)), jnp.float32(-1.0))
    outarr = jnp.zeros((orows, 128), jnp.int32)

    def body(i, st):
        mind, outarr = st
        mx = jnp.max(mind)
        nxt = jnp.min(jnp.where(mind == mx, flat, big_i)).astype(jnp.int32)
        outarr = jnp.where(oflat == i, nxt, outarr)
        return jnp.minimum(mind, dist_to(nxt)), outarr

    _, outarr = jax.lax.fori_loop(1, m, body, (mind, outarr))
    out_ref[...] = outarr


def _fps(pos, m):
    n = pos.shape[0]
    rows = _rup((n + 127) // 128, 8)
    pad = jnp.zeros((3, rows * 128), jnp.float32).at[:, :n].set(pos.T)
    pad = pad.reshape(3, rows, 128)
    orows = _rup((m + 127) // 128, 8)
    out = pl.pallas_call(
        functools.partial(_fps_body, m=m, n=n, rows=rows, orows=orows),
        out_shape=jax.ShapeDtypeStruct((orows, 128), jnp.int32),
    )(pad)
    return out.reshape(-1)[:m]


def _proj_body(x_ref, win_ref, bin_ref, wcat_ref, out_ref):
    h = jnp.maximum(
        jax.lax.dot_general(x_ref[...], win_ref[...], (((1,), (1,)), ((), ())),
                            preferred_element_type=jnp.float32) + bin_ref[...], 0.0)
    out_ref[...] = jax.lax.dot_general(h, wcat_ref[...], (((1,), (1,)), ((), ())),
                                       preferred_element_type=jnp.float32)


def _conv_body(adst_ref, aself_ref, xlself_ref, gsrc_ref, gxl_ref, gpos_ref,
               posq_ref, pw1_ref, pw2_ref, aw1_ref, aw2_ref, wout_ref, bout_ref,
               out_ref, *, blk, dout, kk):
    posq = posq_ref[...]
    rel = posq[:, None, :] - gpos_ref[...]
    pw1t = pw1_ref[...]
    h = (rel[:, :, 0:1] * pw1t[0].reshape(1, 1, 64)
         + rel[:, :, 1:2] * pw1t[1].reshape(1, 1, 64)
         + rel[:, :, 2:3] * pw1t[2].reshape(1, 1, 64))
    h = jnp.maximum(h, 0.0).reshape(blk * kk, 64)
    delta = jnp.maximum(jax.lax.dot_general(h, pw2_ref[...], (((1,), (1,)), ((), ())),
                                            preferred_element_type=jnp.float32), 0.0)
    delta3 = delta.reshape(blk, kk, dout)
    adst = adst_ref[...]
    ain = adst[:, None, :] - gsrc_ref[...] + delta3
    h2 = jnp.maximum(jax.lax.dot_general(ain.reshape(blk * kk, dout), aw1_ref[...],
                                         (((1,), (1,)), ((), ())),
                                         preferred_element_type=jnp.float32), 0.0)
    alpha = jnp.maximum(jax.lax.dot_general(h2, aw2_ref[...], (((1,), (1,)), ((), ())),
                                            preferred_element_type=jnp.float32),
                        0.0).reshape(blk, kk, dout)
    sin = adst - aself_ref[...]
    hs = jnp.maximum(jax.lax.dot_general(sin, aw1_ref[...], (((1,), (1,)), ((), ())),
                                         preferred_element_type=jnp.float32), 0.0)
    alpha_s = jnp.maximum(jax.lax.dot_general(hs, aw2_ref[...], (((1,), (1,)), ((), ())),
                                              preferred_element_type=jnp.float32), 0.0)
    amax = jnp.maximum(jnp.max(alpha, axis=1), alpha_s)
    ex = jnp.exp(alpha - amax[:, None, :])
    ex_s = jnp.exp(alpha_s - amax)
    inv = 1.0 / (jnp.sum(ex, axis=1) + ex_s + 1e-16)
    attn = ex * inv[:, None, :]
    o = (jnp.sum(attn * (gxl_ref[...] + delta3), axis=1)
         + (ex_s * inv) * xlself_ref[...])
    o = jnp.maximum(jax.lax.dot_general(o, wout_ref[...], (((1,), (1,)), ((), ())),
                                        preferred_element_type=jnp.float32)
                    + bout_ref[...], 0.0)
    out_ref[...] = o


def _block_fused(p, x, pos, nbr):
    n, din = x.shape
    c = p["conv"]
    dout = c["lin"].shape[0]
    kk = nbr.shape[1]
    blk = max(128, 65536 // dout)
    n_pad = _rup(n, blk)
    xp = jnp.zeros((n_pad, din), jnp.float32).at[:n].set(x)
    wcat = jnp.concatenate([c["lin"], c["lin_src"], c["lin_dst"]], 0)
    proj = pl.pallas_call(
        _proj_body,
        grid=(n_pad // blk,),
        in_specs=[
            pl.BlockSpec((blk, din), lambda i: (i, 0)),
            pl.BlockSpec((din, din), lambda i: (0, 0)),
            pl.BlockSpec((1, din), lambda i: (0, 0)),
            pl.BlockSpec((3 * dout, din), lambda i: (0, 0)),
        ],
        out_specs=pl.BlockSpec((blk, 3 * dout), lambda i: (i, 0)),
        out_shape=jax.ShapeDtypeStruct((n_pad, 3 * dout), jnp.float32),
    )(xp, p["lin_in"]["w"], p["lin_in"]["b"][None, :], wcat)
    xl = proj[:, :dout]
    a_src = proj[:, dout:2 * dout]
    a_dst = proj[:, 2 * dout:]
    nbr_p = jnp.zeros((n_pad, kk), nbr.dtype).at[:n].set(nbr)
    gsrc = a_src[nbr_p]
    gxl = xl[nbr_p]
    posp = jnp.zeros((n_pad, 3), jnp.float32).at[:n].set(pos)
    gpos = posp[nbr_p]
    pn0, pn1 = c["pos_nn"][0]["lin"], c["pos_nn"][1]["lin"]
    an0, an1 = c["attn_nn"][0]["lin"], c["attn_nn"][1]["lin"]
    out = pl.pallas_call(
        functools.partial(_conv_body, blk=blk, dout=dout, kk=kk),
        grid=(n_pad // blk,),
        in_specs=[
            pl.BlockSpec((blk, dout), lambda i: (i, 0)),
            pl.BlockSpec((blk, dout), lambda i: (i, 0)),
            pl.BlockSpec((blk, dout), lambda i: (i, 0)),
            pl.BlockSpec((blk, kk, dout), lambda i: (i, 0, 0)),
            pl.BlockSpec((blk, kk, dout), lambda i: (i, 0, 0)),
            pl.BlockSpec((blk, kk, 3), lambda i: (i, 0, 0)),
            pl.BlockSpec((blk, 3), lambda i: (i, 0)),
            pl.BlockSpec((3, 64), lambda i: (0, 0)),
            pl.BlockSpec((dout, 64), lambda i: (0, 0)),
            pl.BlockSpec((64, dout), lambda i: (0, 0)),
            pl.BlockSpec((dout, 64), lambda i: (0, 0)),
            pl.BlockSpec((dout, dout), lambda i: (0, 0)),
            pl.BlockSpec((1, dout), lambda i: (0, 0)),
        ],
        out_specs=pl.BlockSpec((blk, dout), lambda i: (i, 0)),
        out_shape=jax.ShapeDtypeStruct((n_pad, dout), jnp.float32),
    )(a_dst, a_src, xl, gsrc, gxl, gpos, posp,
      pn0["w"].T, pn1["w"], an0["w"], an1["w"],
      p["lin_out"]["w"], p["lin_out"]["b"][None, :])
    return out[:n]


def _lin(p, x):
    return x @ p["w"].T + p["b"]


def _bn(p, x):
    mu = jnp.mean(x, 0)
    var = jnp.var(x, 0)
    return p["gamma"] * (x - mu) / jnp.sqrt(var + 1e-5) + p["beta"]


def _mlp_bn(ps, x):
    for p in ps:
        x = jax.nn.relu(_bn(p["bn"], _lin(p["lin"], x)))
    return x


def _mlp_nobn(ps, x):
    for p in ps:
        x = jax.nn.relu(_lin(p["lin"], x))
    return x


def _conv_dense(p, x, pos, nbr):
    n = x.shape[0]
    nbr_full = jnp.concatenate([nbr, jnp.arange(n, dtype=nbr.dtype)[:, None]], 1)
    xl = x @ p["lin"].T
    a_src = x @ p["lin_src"].T
    a_dst = x @ p["lin_dst"].T
    rel = pos[:, None, :] - pos[nbr_full]
    delta = _mlp_nobn(p["pos_nn"], rel)
    alpha = _mlp_nobn(p["attn_nn"], a_dst[:, None, :] - a_src[nbr_full] + delta)
    amax = jnp.max(alpha, axis=1, keepdims=True)
    ex = jnp.exp(alpha - amax)
    den = jnp.sum(ex, axis=1, keepdims=True)
    attn = ex / (den + 1e-16)
    return jnp.sum(attn * (xl[nbr_full] + delta), axis=1)


def _block(p, x, pos, nbr):
    return _block_fused(p, x, pos, nbr)


def _interp(x_sub, pos_sub, pos, k=3):
    nbr = _knn(pos, pos_sub, k, exclude_self=False)
    diff = pos_sub[nbr] - pos[:, None, :]
    sq = jnp.sum(diff * diff, -1, keepdims=True)
    w = 1.0 / jnp.maximum(sq, 1e-16)
    return jnp.sum(x_sub[nbr] * w, axis=1) / jnp.sum(w, axis=1)


def kernel(x, pos, params):
    n0 = pos.shape[0]
    x = _mlp_bn(params["mlp_input"], x)
    nbr0 = _knn(pos, pos, _K, exclude_self=True)
    x = _block(params["transformer_input"], x, pos, nbr0)

    xs, poss, nbrs = [x], [pos], [nbr0]
    for enc in params["encoders"]:
        cur_pos = poss[-1]
        m = int(np.ceil(cur_pos.shape[0] * _RATIO))
        idc = _fps(cur_pos, m)
        nbr_dn = _knn(cur_pos[idc], cur_pos, _K, exclude_self=False)
        xh = _mlp_bn(enc["down"]["mlp"], xs[-1])
        x = jnp.max(xh[nbr_dn], axis=1)
        pos_new = cur_pos[idc]
        nbr = _knn(pos_new, pos_new, _K, exclude_self=True)
        x = _block(enc["block"], x, pos_new, nbr)
        xs.append(x)
        poss.append(pos_new)
        nbrs.append(nbr)

    x = _mlp_nobn(params["mlp_summit"], xs[-1])
    x = _block(params["transformer_summit"], x, poss[-1], nbrs[-1])

    for i, dec in enumerate(params["decoders"]):
        x_skip = xs[-i - 2]
        pos_f, pos_c = poss[-i - 2], poss[-i - 1]
        x_sub = _mlp_bn(dec["up"]["mlp_sub"], x)
        xi = _interp(x_sub, pos_c, pos_f, k=3)
        x = _mlp_bn(dec["up"]["mlp"], x_skip) + xi
        x = _block(dec["block"], x, pos_f, nbrs[-i - 2])

    p0, p1, p2 = params["mlp_output"]
    x = jax.nn.relu(_lin(p0, x))
    x = jax.nn.relu(_lin(p1, x))
    x = _lin(p2, x)
    return jax.nn.log_softmax(x, axis=-1)
